# baseline (speedup 1.0000x reference)
import jax
import jax.numpy as jnp
from jax import lax
from jax.experimental import pallas as pl
from jax.experimental.pallas import tpu as pltpu

F32 = jnp.float32
BF16 = jnp.bfloat16
CP = 128


def _round_up(x, m):
    return (x + m - 1) // m * m


def _full_spec(a):
    nd = a.ndim
    return pl.BlockSpec(a.shape, lambda i, nd=nd: (0,) * nd)


def _conv_taps(a_bf, w_ref, b_ref, p, relu):
    m = a_bf.shape[0]
    k_taps = w_ref.shape[0]
    acc = jnp.zeros((m, CP), F32)
    for k in range(k_taps):
        d = k - p
        shifted = a_bf if d == 0 else pltpu.roll(a_bf, shift=(-d) % m, axis=0)
        acc = acc + jnp.dot(shifted, w_ref[k], preferred_element_type=F32)
    acc = acc + b_ref[...]
    return jnp.maximum(acc, 0.0) if relu else acc


def _row_mask(bt, lslab, h, lv):
    pos = lax.broadcasted_iota(jnp.int32, (bt, lslab, CP), 1).reshape(bt * lslab, CP)
    return (pos >= h) & (pos < h + lv)


def _make_stem_kernel(m_rows):
    def _kernel_body(patch_ref, w_ref, b_ref, o_ref):
        y = jnp.dot(patch_ref[...], w_ref[...], preferred_element_type=F32)
        y = jnp.maximum(y + b_ref[...], 0.0)
        y1 = pltpu.roll(y, shift=(m_rows - 1) % m_rows, axis=0)
        y2 = pltpu.roll(y, shift=(m_rows - 2) % m_rows, axis=0)
        o_ref[...] = jnp.maximum(y, jnp.maximum(y1, y2))
    return _kernel_body


def _make_blocks_kernel(bt, lslab, h, l0):
    def _kernel_body(x_ref,
                     cw1, cb1, cw2, cb2, cw3, cb3, cw4, cb4,
                     j1w1, j1b1, j1w2, j1b2, j1w3, j1b3,
                     j2w1, j2b1, j2w2, j2b2, j2w3, j2b3,
                     o_ref):
        a0 = x_ref[...].reshape(bt * lslab, CP)
        m_l1 = _row_mask(bt, lslab, h, l0 + 1)
        m_l2 = _row_mask(bt, lslab, h, l0 + 2)
        m_l3 = _row_mask(bt, lslab, h, l0 + 3)

        a0b = a0.astype(BF16)
        a1 = jnp.where(m_l1, _conv_taps(a0b, cw1, cb1, 12, True), 0.0)
        a2 = jnp.where(m_l2, _conv_taps(a1.astype(BF16), cw2, cb2, 12, True), 0.0)
        t3 = _conv_taps(a2.astype(BF16), cw3, cb3, 12, True)
        idn = _conv_taps(a0b, cw4, cb4, 12, True)
        y = jnp.where(m_l1, jnp.maximum(t3 + idn, 0.0), 0.0)

        yb = y.astype(BF16)
        a1 = jnp.where(m_l2, _conv_taps(yb, j1w1, j1b1, 6, True), 0.0)
        a2 = jnp.where(m_l3, _conv_taps(a1.astype(BF16), j1w2, j1b2, 6, True), 0.0)
        t3 = _conv_taps(a2.astype(BF16), j1w3, j1b3, 6, False)
        y = jnp.where(m_l1, jnp.maximum(t3 + y, 0.0), 0.0)

        yb = y.astype(BF16)
        a1 = jnp.where(m_l2, _conv_taps(yb, j2w1, j2b1, 3, True), 0.0)
        a2 = jnp.where(m_l3, _conv_taps(a1.astype(BF16), j2w2, j2b2, 3, True), 0.0)
        t3 = _conv_taps(a2.astype(BF16), j2w3, j2b3, 3, False)
        y = jnp.where(m_l1, jnp.maximum(t3 + y, 0.0), 0.0)

        o_ref[...] = y.reshape(bt, lslab, CP)
    return _kernel_body


def _head_kernel(x_ref, w1_ref, b1_ref, w2_ref, o_ref):
    h = jnp.dot(x_ref[...], w1_ref[...], preferred_element_type=F32)
    h = jnp.maximum(h + b1_ref[...], 0.0)
    o_ref[...] = jnp.dot(h.astype(BF16), w2_ref[...],
                         preferred_element_type=F32)[None]


def kernel(x, stem_w, stem_b,
           cb_0, cb_1, cb_2, cb_3, cb_4, cb_5, cb_6, cb_7,
           ib1_0, ib1_1, ib1_2, ib1_3, ib1_4, ib1_5,
           ib2_0, ib2_1, ib2_2, ib2_3, ib2_4, ib2_5,
           fc1_w, fc1_b, fc2_w, fc2_b):
    B, S = x.shape[0], x.shape[2]
    F = 64
    L1 = (S + 6 - 48) // 2 + 1
    L2 = (L1 - 3) // 2 + 1
    bt = 16
    HALO = 12
    lslab = _round_up(2 * HALO + L2 + 3, 8)

    xp = jnp.pad(x[:, 0, :], ((0, 0), (3, 3)))
    idx = 2 * jnp.arange(L1)[:, None] + jnp.arange(48)[None, :]
    patches = xp[:, idx].reshape(B * L1, 48)
    rows_half = (B // 2) * L1
    stem = pl.pallas_call(
        _make_stem_kernel(rows_half),
        out_shape=jax.ShapeDtypeStruct((B * L1, CP), F32),
        grid=(2,),
        in_specs=[pl.BlockSpec((rows_half, 48), lambda i: (i, 0)),
                  _full_spec(stem_w), _full_spec(stem_b)],
        out_specs=pl.BlockSpec((rows_half, CP), lambda i: (i, 0)),
        compiler_params=pltpu.CompilerParams(dimension_semantics=("parallel",)),
    )(patches, stem_w, stem_b)
    y = stem.reshape(B, L1, CP)[:, : 2 * L2 : 2, :]

    slab = jnp.pad(y, ((0, 0), (HALO, lslab - HALO - L2), (0, 0)))
    wbs = [cb_0, cb_1, cb_2, cb_3, cb_4, cb_5, cb_6, cb_7,
           ib1_0, ib1_1, ib1_2, ib1_3, ib1_4, ib1_5,
           ib2_0, ib2_1, ib2_2, ib2_3, ib2_4, ib2_5]
    wbs = [a.astype(BF16) if a.ndim == 3 else a for a in wbs]
    out = pl.pallas_call(
        _make_blocks_kernel(bt, lslab, HALO, L2),
        out_shape=jax.ShapeDtypeStruct((B, lslab, CP), F32),
        grid=(B // bt,),
        in_specs=[pl.BlockSpec((bt, lslab, CP), lambda i: (i, 0, 0))]
                 + [_full_spec(a) for a in wbs],
        out_specs=pl.BlockSpec((bt, lslab, CP), lambda i: (i, 0, 0)),
        compiler_params=pltpu.CompilerParams(dimension_semantics=("parallel",),
                                             vmem_limit_bytes=32 * 1024 * 1024),
    )(slab, *wbs)

    LF = L2 + 1
    feat = jnp.transpose(out[:, HALO:HALO + LF, :F], (0, 2, 1)).reshape(B, F * LF)
    n1h = fc1_w.shape[1] // 2
    partial = pl.pallas_call(
        _head_kernel,
        out_shape=jax.ShapeDtypeStruct((2, B, S), F32),
        grid=(2,),
        in_specs=[pl.BlockSpec((B, F * LF), lambda i: (0, 0)),
                  pl.BlockSpec((F * LF, n1h), lambda i: (0, i)),
                  pl.BlockSpec((1, n1h), lambda i: (0, i)),
                  pl.BlockSpec((n1h, S), lambda i: (i, 0))],
        out_specs=pl.BlockSpec((1, B, S), lambda i: (i, 0, 0)),
        compiler_params=pltpu.CompilerParams(dimension_semantics=("parallel",)),
    )(feat.astype(BF16), fc1_w.astype(BF16), fc1_b, fc2_w.astype(BF16))
    return partial[0] + partial[1] + fc2_b
